# baseline (speedup 1.0000x reference)
import functools

import jax
import jax.numpy as jnp
from jax import lax
from jax.experimental import pallas as pl
from jax.experimental.pallas import tpu as pltpu
from jax.experimental.pallas import tpu_sc as plsc


def _make_sc_row_copy(rows: int, dim: int, chunk: int = 32, nbuf: int = 2):
    info = plsc.get_sparse_core_info()
    num_cores, num_subcores = info.num_cores, info.num_subcores
    num_workers = num_cores * num_subcores
    rows_per_worker = rows // num_workers
    while rows_per_worker % chunk:
        chunk //= 2
    n_chunks = rows_per_worker // chunk
    nbuf = min(nbuf, n_chunks)

    mesh = plsc.VectorSubcoreMesh(core_axis_name="c", subcore_axis_name="s")

    @functools.partial(
        pl.kernel,
        out_type=jax.ShapeDtypeStruct((rows, dim), jnp.float32),
        mesh=mesh,
        scratch_types=(
            [pltpu.VMEM((chunk, dim), jnp.float32)] * nbuf
            + [pltpu.SemaphoreType.DMA] * (2 * nbuf)
        ),
    )
    def copy_kernel(table, out, *refs):
        bufs = refs[:nbuf]
        rsems = refs[nbuf : 2 * nbuf]
        wsems = refs[2 * nbuf :]
        wid = lax.axis_index("s") * num_cores + lax.axis_index("c")
        base = wid * rows_per_worker
        reads = [None] * nbuf
        writes = [None] * nbuf

        def start_read(i):
            b = i % nbuf
            reads[b] = pltpu.make_async_copy(
                table.at[pl.ds(base + i * chunk, chunk)], bufs[b], rsems[b]
            )
            reads[b].start()

        for i in range(nbuf - 1):
            start_read(i)
        for i in range(n_chunks):
            b = i % nbuf
            j = i + nbuf - 1
            if j < n_chunks:
                prev = writes[j % nbuf]
                if prev is not None:
                    prev.wait()
                start_read(j)
            reads[b].wait()
            writes[b] = pltpu.make_async_copy(
                bufs[b], out.at[pl.ds(base + i * chunk, chunk)], wsems[b]
            )
            writes[b].start()
        for i in range(max(0, n_chunks - nbuf), n_chunks):
            writes[i % nbuf].wait()

    return copy_kernel


def _make_sc_row_copy_compact(rows: int, dim: int, chunk: int = 32):
    info = plsc.get_sparse_core_info()
    num_cores, num_subcores = info.num_cores, info.num_subcores
    num_workers = num_cores * num_subcores
    rows_per_worker = rows // num_workers
    while rows_per_worker % (2 * chunk):
        chunk //= 2
    n_chunks = rows_per_worker // chunk
    n_pairs = n_chunks // 2

    mesh = plsc.VectorSubcoreMesh(core_axis_name="c", subcore_axis_name="s")

    @functools.partial(
        pl.kernel,
        out_type=jax.ShapeDtypeStruct((rows, dim), jnp.float32),
        mesh=mesh,
        scratch_types=[
            pltpu.VMEM((chunk, dim), jnp.float32),
            pltpu.VMEM((chunk, dim), jnp.float32),
            pltpu.SemaphoreType.DMA,
            pltpu.SemaphoreType.DMA,
            pltpu.SemaphoreType.DMA,
            pltpu.SemaphoreType.DMA,
        ],
    )
    def copy_kernel(table, out, b0, b1, r0, r1, w0, w1):
        wid = lax.axis_index("s") * num_cores + lax.axis_index("c")
        base = wid * rows_per_worker

        def rd(i, buf, sem):
            return pltpu.make_async_copy(
                table.at[pl.ds(base + i * chunk, chunk)], buf, sem
            )

        def wr(i, buf, sem):
            return pltpu.make_async_copy(
                buf, out.at[pl.ds(base + i * chunk, chunk)], sem
            )

        rd(0, b0, r0).start()

        def body(k, carry):
            i0 = 2 * k

            @pl.when(k > 0)
            def _():
                wr(0, b1, w1).wait()

            rd(i0 + 1, b1, r1).start()
            rd(0, b0, r0).wait()
            wr(i0, b0, w0).start()

            @pl.when(k < n_pairs - 1)
            def _():
                wr(0, b0, w0).wait()
                rd(i0 + 2, b0, r0).start()

            rd(0, b1, r1).wait()
            wr(i0 + 1, b1, w1).start()
            return carry

        lax.fori_loop(0, n_pairs, body, 0)
        wr(0, b0, w0).wait()
        wr(0, b1, w1).wait()

    return copy_kernel


def kernel(x, emb_weight):
    seq = x.shape[1]
    _, dim = emb_weight.shape
    out = _make_sc_row_copy_compact(seq, dim, chunk=32)(emb_weight)
    return out[None]


# speedup vs baseline: 1.0184x; 1.0184x over previous
import functools

import jax
import jax.numpy as jnp
from jax import lax
from jax.experimental import pallas as pl
from jax.experimental.pallas import tpu as pltpu
from jax.experimental.pallas import tpu_sc as plsc


def _make_sc_row_copy(rows: int, dim: int, chunk: int = 32, nbuf: int = 2):
    info = plsc.get_sparse_core_info()
    num_cores, num_subcores = info.num_cores, info.num_subcores
    num_workers = num_cores * num_subcores
    rows_per_worker = rows // num_workers
    while rows_per_worker % chunk:
        chunk //= 2
    n_chunks = rows_per_worker // chunk
    nbuf = min(nbuf, n_chunks)

    mesh = plsc.VectorSubcoreMesh(core_axis_name="c", subcore_axis_name="s")

    @functools.partial(
        pl.kernel,
        out_type=jax.ShapeDtypeStruct((rows, dim), jnp.float32),
        mesh=mesh,
        scratch_types=(
            [pltpu.VMEM((chunk, dim), jnp.float32)] * nbuf
            + [pltpu.SemaphoreType.DMA] * (2 * nbuf)
        ),
    )
    def copy_kernel(table, out, *refs):
        bufs = refs[:nbuf]
        rsems = refs[nbuf : 2 * nbuf]
        wsems = refs[2 * nbuf :]
        wid = lax.axis_index("c") * num_subcores + lax.axis_index("s")
        base = wid * rows_per_worker
        reads = [None] * nbuf
        writes = [None] * nbuf

        def start_read(i):
            b = i % nbuf
            reads[b] = pltpu.make_async_copy(
                table.at[pl.ds(base + i * chunk, chunk)], bufs[b], rsems[b]
            )
            reads[b].start()

        for i in range(nbuf - 1):
            start_read(i)
        for i in range(n_chunks):
            b = i % nbuf
            j = i + nbuf - 1
            if j < n_chunks:
                prev = writes[j % nbuf]
                if prev is not None:
                    prev.wait()
                start_read(j)
            reads[b].wait()
            writes[b] = pltpu.make_async_copy(
                bufs[b], out.at[pl.ds(base + i * chunk, chunk)], wsems[b]
            )
            writes[b].start()
        for i in range(max(0, n_chunks - nbuf), n_chunks):
            writes[i % nbuf].wait()

    return copy_kernel


def _make_sc_row_copy_compact(rows: int, dim: int, chunk: int = 32):
    info = plsc.get_sparse_core_info()
    num_cores, num_subcores = info.num_cores, info.num_subcores
    num_workers = num_cores * num_subcores
    rows_per_worker = rows // num_workers
    while rows_per_worker % (2 * chunk):
        chunk //= 2
    n_chunks = rows_per_worker // chunk
    n_pairs = n_chunks // 2

    mesh = plsc.VectorSubcoreMesh(core_axis_name="c", subcore_axis_name="s")

    @functools.partial(
        pl.kernel,
        out_type=jax.ShapeDtypeStruct((rows, dim), jnp.float32),
        mesh=mesh,
        scratch_types=[
            pltpu.VMEM((chunk, dim), jnp.float32),
            pltpu.VMEM((chunk, dim), jnp.float32),
            pltpu.SemaphoreType.DMA,
            pltpu.SemaphoreType.DMA,
            pltpu.SemaphoreType.DMA,
            pltpu.SemaphoreType.DMA,
        ],
    )
    def copy_kernel(table, out, b0, b1, r0, r1, w0, w1):
        wid = lax.axis_index("s") * num_cores + lax.axis_index("c")
        base = wid * rows_per_worker

        def rd(i, buf, sem):
            return pltpu.make_async_copy(
                table.at[pl.ds(base + i * chunk, chunk)], buf, sem
            )

        def wr(i, buf, sem):
            return pltpu.make_async_copy(
                buf, out.at[pl.ds(base + i * chunk, chunk)], sem
            )

        rd(0, b0, r0).start()

        def body(k, carry):
            i0 = 2 * k

            @pl.when(k > 0)
            def _():
                wr(0, b1, w1).wait()

            rd(i0 + 1, b1, r1).start()
            rd(0, b0, r0).wait()
            wr(i0, b0, w0).start()

            @pl.when(k < n_pairs - 1)
            def _():
                wr(0, b0, w0).wait()
                rd(i0 + 2, b0, r0).start()

            rd(0, b1, r1).wait()
            wr(i0 + 1, b1, w1).start()
            return carry

        lax.fori_loop(0, n_pairs, body, 0)
        wr(0, b0, w0).wait()
        wr(0, b1, w1).wait()

    return copy_kernel


def kernel(x, emb_weight):
    seq = x.shape[1]
    _, dim = emb_weight.shape
    out = _make_sc_row_copy(seq, dim, chunk=32, nbuf=3)(emb_weight)
    return out[None]
